# Initial kernel scaffold; baseline (speedup 1.0000x reference)
import jax
import jax.numpy as jnp
from jax.experimental import pallas as pl


def kernel(x, edge_index, W_l1, b_l1, W_r1, W_l2, b_l2, W_r2, W_out, b_out):
    raise NotImplementedError("write your pallas kernel here")


# speedup vs baseline: 3.7869x; 3.7869x over previous
import jax
import jax.numpy as jnp
from jax import lax
from jax.experimental import pallas as pl
from jax.experimental.pallas import tpu as pltpu
from jax.experimental.pallas import tpu_sc as plsc

N = 10000
E = 160000
D = 256
Q = 64
NS = 16
EPT = E // NS
K = 40
NCH = EPT // K
NPAD = 10240
RPT = NPAD // NS
BM = 1000

_f32 = jnp.float32
_row2 = lambda i: (i, 0)
_fix2 = lambda i: (0, 0)


def _dotT(a, w):
    return lax.dot_general(a, w, (((1,), (1,)), ((), ())),
                           preferred_element_type=jnp.float32)


def _split_q(y, refs):
    for q, ref in enumerate(refs):
        ref[...] = y[:, q * Q:(q + 1) * Q]


def _mm_first_body(x_ref, wl_ref, wr_ref, bl_ref,
                   y0_ref, y1_ref, y2_ref, y3_ref, z_ref):
    xb = x_ref[...]
    _split_q(_dotT(xb, wl_ref[...]), (y0_ref, y1_ref, y2_ref, y3_ref))
    z_ref[...] = _dotT(xb, wr_ref[...]) + bl_ref[...]


def _agg_h(a_refs, cnt_ref, z_ref):
    inv = 1.0 / jnp.maximum(cnt_ref[...], 1.0)
    agg = jnp.concatenate([a[...] for a in a_refs], axis=1) * inv
    return jnp.maximum(agg + z_ref[...], 0.0)


def _mm_mid_body(a0, a1, a2, a3, cnt_ref, z_ref, wl_ref, wr_ref, bl_ref,
                 y0_ref, y1_ref, y2_ref, y3_ref, z2_ref):
    h = _agg_h((a0, a1, a2, a3), cnt_ref, z_ref)
    _split_q(_dotT(h, wl_ref[...]), (y0_ref, y1_ref, y2_ref, y3_ref))
    z2_ref[...] = _dotT(h, wr_ref[...]) + bl_ref[...]


def _mm_out_body(a0, a1, a2, a3, cnt_ref, z_ref, wo_ref, bo_ref, out_ref):
    h = _agg_h((a0, a1, a2, a3), cnt_ref, z_ref)
    out_ref[...] = jnp.sum(h * wo_ref[...], axis=1, keepdims=True) + bo_ref[0, 0]


def _mm_first(x, wl, wr, bl):
    return pl.pallas_call(
        _mm_first_body,
        grid=(N // BM,),
        in_specs=[
            pl.BlockSpec((BM, D), _row2),
            pl.BlockSpec((D, D), _fix2),
            pl.BlockSpec((D, D), _fix2),
            pl.BlockSpec((1, D), _fix2),
        ],
        out_specs=[pl.BlockSpec((BM, Q), _row2)] * 4
        + [pl.BlockSpec((BM, D), _row2)],
        out_shape=[jax.ShapeDtypeStruct((N, Q), _f32)] * 4
        + [jax.ShapeDtypeStruct((N, D), _f32)],
    )(x, wl, wr, bl)


def _mm_mid(aggs, cnt2d, z, wl, wr, bl):
    return pl.pallas_call(
        _mm_mid_body,
        grid=(N // BM,),
        in_specs=[pl.BlockSpec((BM, Q), _row2)] * 4 + [
            pl.BlockSpec((BM, 1), _row2),
            pl.BlockSpec((BM, D), _row2),
            pl.BlockSpec((D, D), _fix2),
            pl.BlockSpec((D, D), _fix2),
            pl.BlockSpec((1, D), _fix2),
        ],
        out_specs=[pl.BlockSpec((BM, Q), _row2)] * 4
        + [pl.BlockSpec((BM, D), _row2)],
        out_shape=[jax.ShapeDtypeStruct((N, Q), _f32)] * 4
        + [jax.ShapeDtypeStruct((N, D), _f32)],
    )(*aggs, cnt2d, z, wl, wr, bl)


def _mm_out(aggs, cnt2d, z, wo, bo):
    return pl.pallas_call(
        _mm_out_body,
        grid=(N // BM,),
        in_specs=[pl.BlockSpec((BM, Q), _row2)] * 4 + [
            pl.BlockSpec((BM, 1), _row2),
            pl.BlockSpec((BM, D), _row2),
            pl.BlockSpec((1, D), _fix2),
            pl.BlockSpec((1, D), _fix2),
        ],
        out_specs=[pl.BlockSpec((BM, 1), _row2)],
        out_shape=[jax.ShapeDtypeStruct((N, 1), _f32)],
    )(*aggs, cnt2d, z, wo, bo)


def _sc_body(compute_cnt, y0_hbm, y1_hbm, y2_hbm, y3_hbm, src_hbm, dst_hbm,
             o0, o1, o2, o3, cnt_hbm,
             src_v, dst_v, gbuf0, gbuf1, zbuf, zc, ones_v,
             acc_sh, cnt_sh, sem0, sem1):
    cid = lax.axis_index("c")
    sid = lax.axis_index("s")
    z16 = jnp.zeros((16,), _f32)

    @pl.loop(0, zbuf.shape[0])
    def _(r):
        for c in range(Q // 16):
            zbuf[r, pl.ds(c * 16, 16)] = z16

    def _zero_acc():
        zr = zbuf.shape[0]
        for j in range(RPT // zr):
            pltpu.sync_copy(zbuf, acc_sh.at[pl.ds(sid * RPT + j * zr, zr)])

    _zero_acc()

    if compute_cnt:
        @pl.loop(0, zc.shape[0])
        def _(r):
            zc[r, :] = z16

        @pl.loop(0, K)
        def _(r):
            ones_v[r, :] = z16 + 1.0

        @pl.when(cid == 0)
        def _():
            zr = zc.shape[0]
            for j in range(RPT // zr):
                pltpu.sync_copy(zc, cnt_sh.at[pl.ds(sid * RPT + j * zr, zr)])

    pltpu.sync_copy(src_hbm.at[sid], src_v)
    pltpu.sync_copy(dst_hbm.at[sid], dst_v)

    plsc.subcore_barrier()

    def _run(tbl, do_cnt):
        pltpu.async_copy(tbl.at[src_v.at[0]], gbuf0, sem0)

        def _scatter(g, j):
            pltpu.sync_copy(g, acc_sh.at[dst_v.at[j]], add=True)
            if do_cnt:
                pltpu.sync_copy(ones_v, cnt_sh.at[dst_v.at[j]], add=True)

        @pl.loop(0, NCH // 2)
        def _(t):
            j = 2 * t
            pltpu.async_copy(tbl.at[src_v.at[j + 1]], gbuf1, sem1)
            pltpu.make_async_copy(tbl.at[src_v.at[j]], gbuf0, sem0).wait()
            _scatter(gbuf0, j)

            @pl.when(t < NCH // 2 - 1)
            def _():
                pltpu.async_copy(tbl.at[src_v.at[j + 2]], gbuf0, sem0)

            pltpu.make_async_copy(tbl.at[src_v.at[j + 1]], gbuf1, sem1).wait()
            _scatter(gbuf1, j + 1)

    @pl.when(cid == 0)
    def _():
        _run(y0_hbm, compute_cnt)

    @pl.when(cid == 1)
    def _():
        _run(y2_hbm, False)

    plsc.subcore_barrier()

    @pl.when(cid == 0)
    def _():
        pltpu.sync_copy(acc_sh.at[pl.ds(sid * RPT, RPT)], o0.at[sid])

    @pl.when(cid == 1)
    def _():
        pltpu.sync_copy(acc_sh.at[pl.ds(sid * RPT, RPT)], o2.at[sid])

    if compute_cnt:
        @pl.when(cid == 0)
        def _():
            pltpu.sync_copy(cnt_sh.at[pl.ds(sid * RPT, RPT)], cnt_hbm.at[sid])

    _zero_acc()
    plsc.subcore_barrier()

    @pl.when(cid == 0)
    def _():
        _run(y1_hbm, False)

    @pl.when(cid == 1)
    def _():
        _run(y3_hbm, False)

    plsc.subcore_barrier()

    @pl.when(cid == 0)
    def _():
        pltpu.sync_copy(acc_sh.at[pl.ds(sid * RPT, RPT)], o1.at[sid])

    @pl.when(cid == 1)
    def _():
        pltpu.sync_copy(acc_sh.at[pl.ds(sid * RPT, RPT)], o3.at[sid])


def _make_sc(compute_cnt):
    mesh = plsc.VectorSubcoreMesh(core_axis_name="c", subcore_axis_name="s")
    out_type = [jax.ShapeDtypeStruct((NS, RPT, Q), _f32)] * 4
    if compute_cnt:
        out_type.append(jax.ShapeDtypeStruct((NS, RPT, 16), _f32))
    scratch = [
        pltpu.VMEM((NCH, K), jnp.int32),
        pltpu.VMEM((NCH, K), jnp.int32),
        pltpu.VMEM((K, Q), _f32),
        pltpu.VMEM((K, Q), _f32),
        pltpu.VMEM((128, Q), _f32),
        pltpu.VMEM((128, 16), _f32),
        pltpu.VMEM((K, 16), _f32),
        pltpu.VMEM_SHARED((NPAD, Q), _f32),
        pltpu.VMEM_SHARED((NPAD, 16), _f32),
        pltpu.SemaphoreType.DMA,
        pltpu.SemaphoreType.DMA,
    ]

    if compute_cnt:
        def body(y0, y1, y2, y3, src3, dst3, o0, o1, o2, o3, cnt, *scr):
            _sc_body(True, y0, y1, y2, y3, src3, dst3,
                     o0, o1, o2, o3, cnt, *scr)
    else:
        def body(y0, y1, y2, y3, src3, dst3, o0, o1, o2, o3, *scr):
            _sc_body(False, y0, y1, y2, y3, src3, dst3,
                     o0, o1, o2, o3, None, *scr)

    return pl.kernel(body, out_type=out_type, mesh=mesh, scratch_types=scratch,
                     compiler_params=pltpu.CompilerParams(
                         use_tc_tiling_on_sc=False))


def _trim(a):
    return a.reshape(NPAD, Q)[:N]


def kernel(x, edge_index, W_l1, b_l1, W_r1, W_l2, b_l2, W_r2, W_out, b_out):
    ei = edge_index.astype(jnp.int32)
    src3 = ei[0].reshape(NS, NCH, K)
    dst3 = ei[1].reshape(NS, NCH, K)
    bl1 = b_l1.reshape(1, D)
    bl2 = b_l2.reshape(1, D)
    bo = jnp.broadcast_to(b_out.reshape(1, 1), (1, D))

    sc_first = _make_sc(True)
    sc_second = _make_sc(False)

    y0, y1, y2, y3, z1 = _mm_first(x, W_l1, W_r1, bl1)
    a0, a1, a2, a3, cnt = sc_first(y0, y1, y2, y3, src3, dst3)
    aggs1 = [_trim(a) for a in (a0, a1, a2, a3)]
    cnt2d = cnt.reshape(NPAD, 16)[:N, 0:1]
    u0, u1, u2, u3, z2 = _mm_mid(aggs1, cnt2d, z1, W_l2, W_r2, bl2)
    b0, b1, b2, b3 = sc_second(u0, u1, u2, u3, src3, dst3)
    aggs2 = [_trim(b) for b in (b0, b1, b2, b3)]
    (logits,) = _mm_out(aggs2, cnt2d, z2, W_out, bo)
    return logits.reshape(N)
